# Initial kernel scaffold; baseline (speedup 1.0000x reference)
import jax
import jax.numpy as jnp
from jax.experimental import pallas as pl


def kernel(x, edge_index, batch, gamma, beta, W1l, b1, W1r, W2l, b2, W2r, Cw1, Cb1, Cw2, Cb2):
    raise NotImplementedError("write your pallas kernel here")


# speedup vs baseline: 7.4359x; 7.4359x over previous
import functools

import jax
import jax.numpy as jnp
from jax import lax
from jax.experimental import pallas as pl
from jax.experimental.pallas import tpu as pltpu
from jax.experimental.pallas import tpu_sc as plsc

N = 50000
E = 800000
F_IN = 3
H = 128
H4 = 32
OUT = 10
G = 64

NC = 2
NS = 16
NW = NC * NS

N_PAD = 50176
E_PAD = 819200
K_CHUNKS = E_PAD // (NW * 128)
ROWS_PER_TILE = N_PAD // NS
IDXBLK = 25
N_IBLK = K_CHUNKS // IDXBLK


def _leaky(v):
    return jnp.where(v > 0, v, 0.01 * v)


@functools.lru_cache(maxsize=None)
def _make_edge_scatter(d):
    mesh = plsc.VectorSubcoreMesh(core_axis_name="c", subcore_axis_name="s",
                                  num_cores=NC, num_subcores=NS)

    def body(val_hbm, src_hbm, dst_hbm, zeros_hbm, out_hbm,
             src_v, dst_v, rows_v, acc_sh, sem):
        c = lax.axis_index("c")
        s = lax.axis_index("s")
        w = s * NC + c
        r0 = s * ROWS_PER_TILE
        pltpu.sync_copy(zeros_hbm.at[pl.ds(r0, ROWS_PER_TILE)],
                        acc_sh.at[pl.ds(r0, ROWS_PER_TILE)])
        plsc.subcore_barrier()

        @pl.loop(0, N_IBLK)
        def _blk(b):
            pltpu.sync_copy(src_hbm.at[w, pl.ds(b * IDXBLK, IDXBLK)], src_v)
            pltpu.sync_copy(dst_hbm.at[w, pl.ds(b * IDXBLK, IDXBLK)], dst_v)

            @pl.loop(0, IDXBLK)
            def _chunk(j):
                pltpu.async_copy(val_hbm.at[src_v.at[j]], rows_v, sem).wait()
                pltpu.sync_copy(rows_v, acc_sh.at[dst_v.at[j]], add=True)

        plsc.subcore_barrier()
        pltpu.sync_copy(acc_sh.at[pl.ds(r0, ROWS_PER_TILE)],
                        out_hbm.at[c, pl.ds(r0, ROWS_PER_TILE)])

    return pl.kernel(
        body,
        out_type=jax.ShapeDtypeStruct((NC, N_PAD, d), jnp.float32),
        mesh=mesh,
        compiler_params=pltpu.CompilerParams(use_tc_tiling_on_sc=False),
        scratch_types=[
            pltpu.VMEM((IDXBLK, 128), jnp.int32),
            pltpu.VMEM((IDXBLK, 128), jnp.int32),
            pltpu.VMEM((128, d), jnp.float32),
            pltpu.VMEM_SHARED((N_PAD, d), jnp.float32),
            pltpu.SemaphoreType.DMA,
        ],
    )


def _segsum_sc(val, src_i, dst_i, zeros, d):
    return _make_edge_scatter(d)(val, src_i, dst_i, zeros)


def _prep_body(x_ref, g_ref, b_ref, out_ref):
    xb = x_ref[...]
    mean = jnp.sum(xb, axis=0, keepdims=True) * (1.0 / N)
    sq = jnp.sum(xb * xb, axis=0, keepdims=True) * (1.0 / N)
    var = sq - mean * mean
    inv = lax.rsqrt(var + 1e-5)
    out_ref[...] = (xb - mean) * inv * g_ref[...] + b_ref[...]


def _prep(x8, gamma8, beta8):
    return pl.pallas_call(
        _prep_body,
        out_shape=jax.ShapeDtypeStruct((N_PAD, 8), jnp.float32),
    )(x8, gamma8, beta8)


_BB = 1792
_NB = N_PAD // _BB


def _layer1_body(acc1_ref, val1_ref, w1l_ref, w1r_ref, b1_ref,
                 w2l_ref, w2r_ref, b2_ref, val2_ref, h1r_ref):
    s = acc1_ref[0] + acc1_ref[1]
    deg = jnp.maximum(s[:, 3:4], 1.0)
    t = s / deg
    v1 = val1_ref[...]
    h1 = _leaky(
        jnp.dot(t, w1l_ref[...], preferred_element_type=jnp.float32)
        + b1_ref[...]
        + jnp.dot(v1, w1r_ref[...], preferred_element_type=jnp.float32))
    val2_ref[...] = jnp.dot(h1, w2l_ref[...], preferred_element_type=jnp.float32)
    h1r_ref[...] = (
        jnp.dot(h1, w2r_ref[...], preferred_element_type=jnp.float32)
        + b2_ref[...])


def _layer1(acc1, val1, w1l16, w1r16, b1, w2l, w2r, b2):
    return pl.pallas_call(
        _layer1_body,
        grid=(_NB,),
        in_specs=[
            pl.BlockSpec((NC, _BB, 8), lambda i: (0, i, 0)),
            pl.BlockSpec((_BB, 8), lambda i: (i, 0)),
            pl.BlockSpec((8, H), lambda i: (0, 0)),
            pl.BlockSpec((8, H), lambda i: (0, 0)),
            pl.BlockSpec((1, H), lambda i: (0, 0)),
            pl.BlockSpec((H, H4), lambda i: (0, 0)),
            pl.BlockSpec((H, H4), lambda i: (0, 0)),
            pl.BlockSpec((1, H4), lambda i: (0, 0)),
        ],
        out_specs=[
            pl.BlockSpec((_BB, H4), lambda i: (i, 0)),
            pl.BlockSpec((_BB, H4), lambda i: (i, 0)),
        ],
        out_shape=[
            jax.ShapeDtypeStruct((N_PAD, H4), jnp.float32),
            jax.ShapeDtypeStruct((N_PAD, H4), jnp.float32),
        ],
    )(acc1, val1, w1l16, w1r16, b1, w2l, w2r, b2)


_BC = 256
_NCBLK = N_PAD // _BC


def _final_body(acc2_ref, acc1_ref, h1r_ref, batch_ref,
                cw1_ref, cb1_ref, cw2_ref, cb2_ref, out_ref, pool_ref):
    i = pl.program_id(0)
    s3 = acc1_ref[0, :, 3:4] + acc1_ref[1, :, 3:4]
    deg = jnp.maximum(s3, 1.0)
    agg = (acc2_ref[0] + acc2_ref[1]) / deg
    h2 = _leaky(agg + h1r_ref[...])
    bid = batch_ref[...]

    rows = []
    for g in range(G):
        sel = bid == g
        rows.append(jnp.max(jnp.where(sel, h2, -1e30), axis=0, keepdims=True))
    blockmax = jnp.concatenate(rows, axis=0)

    @pl.when(i == 0)
    def _():
        pool_ref[...] = blockmax

    @pl.when(i > 0)
    def _():
        pool_ref[...] = jnp.maximum(pool_ref[...], blockmax)

    @pl.when(i == _NCBLK - 1)
    def _():
        pooled = pool_ref[...]
        z1 = _leaky(
            jnp.dot(pooled, cw1_ref[...], preferred_element_type=jnp.float32)
            + cb1_ref[...])
        z = (jnp.dot(z1, cw2_ref[...], preferred_element_type=jnp.float32)
             + cb2_ref[...])
        m = jnp.max(z, axis=1, keepdims=True)
        e = jnp.exp(z - m)
        out_ref[...] = e / jnp.sum(e, axis=1, keepdims=True)


def _final(acc2, acc1, h1r, batchcol, cw1, cb1, cw2, cb2):
    return pl.pallas_call(
        _final_body,
        grid=(_NCBLK,),
        in_specs=[
            pl.BlockSpec((NC, _BC, H4), lambda i: (0, i, 0)),
            pl.BlockSpec((NC, _BC, 8), lambda i: (0, i, 0)),
            pl.BlockSpec((_BC, H4), lambda i: (i, 0)),
            pl.BlockSpec((_BC, 1), lambda i: (i, 0)),
            pl.BlockSpec((H4, H4), lambda i: (0, 0)),
            pl.BlockSpec((1, H4), lambda i: (0, 0)),
            pl.BlockSpec((H4, OUT), lambda i: (0, 0)),
            pl.BlockSpec((1, OUT), lambda i: (0, 0)),
        ],
        out_specs=pl.BlockSpec((G, OUT), lambda i: (0, 0)),
        out_shape=jax.ShapeDtypeStruct((G, OUT), jnp.float32),
        scratch_shapes=[pltpu.VMEM((G, H4), jnp.float32)],
    )(acc2, acc1, h1r, batchcol, cw1, cb1, cw2, cb2)


def kernel(x, edge_index, batch, gamma, beta, W1l, b1, W1r, W2l, b2, W2r,
           Cw1, Cb1, Cw2, Cb2):
    f32 = jnp.float32
    x8 = jnp.zeros((N_PAD, 8), f32).at[:N, :F_IN].set(x)
    gamma8 = jnp.zeros((1, 8), f32).at[0, :F_IN].set(gamma)
    beta8 = jnp.zeros((1, 8), f32).at[0, :F_IN].set(beta).at[0, 3].set(1.0)

    npad_e = E_PAD - E
    fill = N + (jnp.arange(npad_e, dtype=jnp.int32) % 128)
    src_i = jnp.concatenate([edge_index[0], fill]).reshape(NW, K_CHUNKS, 128)
    dst_i = jnp.concatenate([edge_index[1], fill]).reshape(NW, K_CHUNKS, 128)

    z8 = jnp.zeros((N_PAD, 8), f32)
    z32 = jnp.zeros((N_PAD, H4), f32)

    w1l8 = jnp.zeros((8, H), f32).at[:F_IN].set(W1l)
    w1r8 = jnp.zeros((8, H), f32).at[:F_IN].set(W1r)
    b1r = b1.reshape(1, H)
    b2r = b2.reshape(1, H4)
    cb1r = Cb1.reshape(1, H4)
    cb2r = Cb2.reshape(1, OUT)

    batchcol = jnp.concatenate(
        [batch, jnp.full((N_PAD - N,), G, jnp.int32)]).reshape(N_PAD, 1)

    val1 = _prep(x8, gamma8, beta8)
    acc1 = _segsum_sc(val1, src_i, dst_i, z8, 8)
    val2, h1r = _layer1(acc1, val1, w1l8, w1r8, b1r, W2l, W2r, b2r)
    acc2 = _segsum_sc(val2, src_i, dst_i, z32, H4)
    return _final(acc2, acc1, h1r, batchcol, Cw1, cb1r, Cw2, cb2r)


# speedup vs baseline: 13.5878x; 1.8273x over previous
import functools

import jax
import jax.numpy as jnp
from jax import lax
from jax.experimental import pallas as pl
from jax.experimental.pallas import tpu as pltpu
from jax.experimental.pallas import tpu_sc as plsc

N = 50000
E = 800000
F_IN = 3
H = 128
H4 = 32
OUT = 10
G = 64

NC = 2
NS = 16
NW = NC * NS

N_PAD = 50176
E_PAD = 819200
K_CHUNKS = E_PAD // (NW * 128)
ROWS_PER_TILE = N_PAD // NS
IDXBLK = 25
N_IBLK = K_CHUNKS // IDXBLK


def _leaky(v):
    return jnp.where(v > 0, v, 0.01 * v)


@functools.lru_cache(maxsize=None)
def _make_edge_scatter(d):
    mesh = plsc.VectorSubcoreMesh(core_axis_name="c", subcore_axis_name="s",
                                  num_cores=NC, num_subcores=NS)

    def body(val_hbm, src_hbm, dst_hbm, zeros_hbm, out_hbm,
             src_v, dst_v, rows_v, acc_sh, sem0, sem1):
        c = lax.axis_index("c")
        s = lax.axis_index("s")
        w = s * NC + c
        r0 = s * ROWS_PER_TILE
        pltpu.sync_copy(zeros_hbm.at[pl.ds(r0, ROWS_PER_TILE)],
                        acc_sh.at[pl.ds(r0, ROWS_PER_TILE)])
        plsc.subcore_barrier()

        rows0 = rows_v.at[0]
        rows1 = rows_v.at[1]

        @pl.loop(0, N_IBLK)
        def _blk(b):
            pltpu.sync_copy(src_hbm.at[w, pl.ds(b * IDXBLK, IDXBLK)], src_v)
            pltpu.sync_copy(dst_hbm.at[w, pl.ds(b * IDXBLK, IDXBLK)], dst_v)
            pltpu.async_copy(val_hbm.at[src_v.at[0]], rows0, sem0)

            @pl.loop(0, (IDXBLK - 1) // 2)
            def _steady(u):
                j = 2 * u
                pltpu.async_copy(val_hbm.at[src_v.at[j + 1]], rows1, sem1)
                pltpu.make_async_copy(val_hbm.at[src_v.at[j]], rows0, sem0).wait()
                pltpu.sync_copy(rows0, acc_sh.at[dst_v.at[j]], add=True)
                pltpu.async_copy(val_hbm.at[src_v.at[j + 2]], rows0, sem0)
                pltpu.make_async_copy(val_hbm.at[src_v.at[j + 1]], rows1, sem1).wait()
                pltpu.sync_copy(rows1, acc_sh.at[dst_v.at[j + 1]], add=True)

            pltpu.make_async_copy(val_hbm.at[src_v.at[IDXBLK - 1]], rows0,
                                  sem0).wait()
            pltpu.sync_copy(rows0, acc_sh.at[dst_v.at[IDXBLK - 1]], add=True)

        plsc.subcore_barrier()
        pltpu.sync_copy(acc_sh.at[pl.ds(r0, ROWS_PER_TILE)],
                        out_hbm.at[c, pl.ds(r0, ROWS_PER_TILE)])

    return pl.kernel(
        body,
        out_type=jax.ShapeDtypeStruct((NC, N_PAD, d), jnp.float32),
        mesh=mesh,
        compiler_params=pltpu.CompilerParams(use_tc_tiling_on_sc=False),
        scratch_types=[
            pltpu.VMEM((IDXBLK, 128), jnp.int32),
            pltpu.VMEM((IDXBLK, 128), jnp.int32),
            pltpu.VMEM((2, 128, d), jnp.float32),
            pltpu.VMEM_SHARED((N_PAD, d), jnp.float32),
            pltpu.SemaphoreType.DMA,
            pltpu.SemaphoreType.DMA,
        ],
    )


def _segsum_sc(val, src_i, dst_i, zeros, d):
    return _make_edge_scatter(d)(val, src_i, dst_i, zeros)


def _prep_body(x_ref, g_ref, b_ref, out_ref):
    xb = x_ref[...]
    mean = jnp.sum(xb, axis=0, keepdims=True) * (1.0 / N)
    sq = jnp.sum(xb * xb, axis=0, keepdims=True) * (1.0 / N)
    var = sq - mean * mean
    inv = lax.rsqrt(var + 1e-5)
    out_ref[...] = (xb - mean) * inv * g_ref[...] + b_ref[...]


def _prep(x8, gamma8, beta8):
    return pl.pallas_call(
        _prep_body,
        out_shape=jax.ShapeDtypeStruct((N_PAD, 8), jnp.float32),
    )(x8, gamma8, beta8)


_BB = 1792
_NB = N_PAD // _BB


def _layer1_body(acc1_ref, val1_ref, w1l_ref, w1r_ref, b1_ref,
                 w2l_ref, w2r_ref, b2_ref, val2_ref, h1r_ref):
    s = acc1_ref[0] + acc1_ref[1]
    deg = jnp.maximum(s[:, 3:4], 1.0)
    t = s / deg
    v1 = val1_ref[...]
    h1 = _leaky(
        jnp.dot(t, w1l_ref[...], preferred_element_type=jnp.float32)
        + b1_ref[...]
        + jnp.dot(v1, w1r_ref[...], preferred_element_type=jnp.float32))
    val2_ref[...] = jnp.dot(h1, w2l_ref[...], preferred_element_type=jnp.float32)
    h1r_ref[...] = (
        jnp.dot(h1, w2r_ref[...], preferred_element_type=jnp.float32)
        + b2_ref[...])


def _layer1(acc1, val1, w1l16, w1r16, b1, w2l, w2r, b2):
    return pl.pallas_call(
        _layer1_body,
        grid=(_NB,),
        in_specs=[
            pl.BlockSpec((NC, _BB, 8), lambda i: (0, i, 0)),
            pl.BlockSpec((_BB, 8), lambda i: (i, 0)),
            pl.BlockSpec((8, H), lambda i: (0, 0)),
            pl.BlockSpec((8, H), lambda i: (0, 0)),
            pl.BlockSpec((1, H), lambda i: (0, 0)),
            pl.BlockSpec((H, H4), lambda i: (0, 0)),
            pl.BlockSpec((H, H4), lambda i: (0, 0)),
            pl.BlockSpec((1, H4), lambda i: (0, 0)),
        ],
        out_specs=[
            pl.BlockSpec((_BB, H4), lambda i: (i, 0)),
            pl.BlockSpec((_BB, H4), lambda i: (i, 0)),
        ],
        out_shape=[
            jax.ShapeDtypeStruct((N_PAD, H4), jnp.float32),
            jax.ShapeDtypeStruct((N_PAD, H4), jnp.float32),
        ],
    )(acc1, val1, w1l16, w1r16, b1, w2l, w2r, b2)


_BC = 512
_NCBLK = N_PAD // _BC


def _final_body(acc2_ref, acc1_ref, h1r_ref, batch_ref,
                cw1_ref, cb1_ref, cw2_ref, cb2_ref, seg_ref, out_ref, pool_ref):
    i = pl.program_id(0)

    @pl.when(i == 0)
    def _():
        pool_ref[...] = jnp.full((16, 128), -1e30, jnp.float32)

    s3 = acc1_ref[0, :, 3:4] + acc1_ref[1, :, 3:4]
    deg = jnp.maximum(s3, 1.0)
    agg = (acc2_ref[0] + acc2_ref[1]) / deg
    h2 = _leaky(agg + h1r_ref[...])
    bid = batch_ref[...]

    b0 = batch_ref[0, 0]
    bl = batch_ref[_BC - 1, 0]
    lq = lax.broadcasted_iota(jnp.int32, (1, 128), 1) // 32

    @pl.when(b0 == bl)
    def _():
        bm = jnp.max(h2, axis=0, keepdims=True)
        bm4 = jnp.concatenate([bm, bm, bm, bm], axis=1)
        rowm = lax.broadcasted_iota(jnp.int32, (16, 1), 0) == b0 // 4
        m = jnp.logical_and(rowm, lq == b0 % 4)
        pool_ref[...] = jnp.where(m, jnp.maximum(pool_ref[...], bm4),
                                  pool_ref[...])

    @pl.when(b0 != bl)
    def _():
        h4 = jnp.concatenate([h2, h2, h2, h2], axis=1)
        rows = []
        for t in range(16):
            sel = bid == 4 * t + lq
            rows.append(jnp.max(jnp.where(sel, h4, -1e30), axis=0,
                                keepdims=True))
        pool_ref[...] = jnp.maximum(pool_ref[...],
                                    jnp.concatenate(rows, axis=0))

    @pl.when(i == _NCBLK - 1)
    def _():
        pooled = pool_ref[...]
        z1 = _leaky(
            jnp.dot(pooled, cw1_ref[...], preferred_element_type=jnp.float32)
            + cb1_ref[...])
        z = (jnp.dot(z1, cw2_ref[...], preferred_element_type=jnp.float32)
             + cb2_ref[...])
        lane16 = lax.broadcasted_iota(jnp.int32, (1, 64), 1) % 16
        zm = jnp.where(lane16 < OUT, z, -1e30)
        e = jnp.exp(zm - jnp.max(zm))
        s = jnp.dot(e, seg_ref[...], preferred_element_type=jnp.float32)
        out_ref[...] = e / s


def _final(acc2, acc1, h1r, batchcol, cw1d, cb1d, cw2d, cb2d, seg):
    return pl.pallas_call(
        _final_body,
        grid=(_NCBLK,),
        in_specs=[
            pl.BlockSpec((NC, _BC, H4), lambda i: (0, i, 0)),
            pl.BlockSpec((NC, _BC, 8), lambda i: (0, i, 0)),
            pl.BlockSpec((_BC, H4), lambda i: (i, 0)),
            pl.BlockSpec((_BC, 1), lambda i: (i, 0)),
            pl.BlockSpec((128, 128), lambda i: (0, 0)),
            pl.BlockSpec((1, 128), lambda i: (0, 0)),
            pl.BlockSpec((128, 64), lambda i: (0, 0)),
            pl.BlockSpec((1, 64), lambda i: (0, 0)),
            pl.BlockSpec((64, 64), lambda i: (0, 0)),
        ],
        out_specs=pl.BlockSpec((16, 64), lambda i: (0, 0)),
        out_shape=jax.ShapeDtypeStruct((16, 64), jnp.float32),
        scratch_shapes=[pltpu.VMEM((16, 128), jnp.float32)],
    )(acc2, acc1, h1r, batchcol, cw1d, cb1d, cw2d, cb2d, seg)


def kernel(x, edge_index, batch, gamma, beta, W1l, b1, W1r, W2l, b2, W2r,
           Cw1, Cb1, Cw2, Cb2):
    f32 = jnp.float32
    x8 = jnp.zeros((N_PAD, 8), f32).at[:N, :F_IN].set(x)
    gamma8 = jnp.zeros((1, 8), f32).at[0, :F_IN].set(gamma)
    beta8 = jnp.zeros((1, 8), f32).at[0, :F_IN].set(beta).at[0, 3].set(1.0)

    npad_e = E_PAD - E
    fill = N + (jnp.arange(npad_e, dtype=jnp.int32) % 128)
    src_i = jnp.concatenate([edge_index[0], fill]).reshape(NW, K_CHUNKS, 128)
    dst_i = jnp.concatenate([edge_index[1], fill]).reshape(NW, K_CHUNKS, 128)

    z8 = jnp.zeros((N_PAD, 8), f32)
    z32 = jnp.zeros((N_PAD, H4), f32)

    w1l8 = jnp.zeros((8, H), f32).at[:F_IN].set(W1l)
    w1r8 = jnp.zeros((8, H), f32).at[:F_IN].set(W1r)
    b1r = b1.reshape(1, H)
    b2r = b2.reshape(1, H4)

    cw1d = jnp.zeros((128, 128), f32)
    cw2d = jnp.zeros((128, 64), f32)
    for q in range(4):
        cw1d = cw1d.at[q * 32:(q + 1) * 32, q * 32:(q + 1) * 32].set(Cw1)
        cw2d = cw2d.at[q * 32:(q + 1) * 32, q * 16:q * 16 + OUT].set(Cw2)
    cb1d = jnp.tile(Cb1.reshape(1, H4), (1, 4))
    cb2d = jnp.tile(
        jnp.concatenate([Cb2, jnp.zeros((16 - OUT,), f32)]).reshape(1, 16),
        (1, 4))
    seg16 = (jnp.arange(64)[:, None] // 16 == jnp.arange(64)[None, :] // 16
             ).astype(f32)

    batchcol = jnp.concatenate(
        [batch, jnp.full((N_PAD - N,), G, jnp.int32)]).reshape(N_PAD, 1)

    val1 = _prep(x8, gamma8, beta8)
    acc1 = _segsum_sc(val1, src_i, dst_i, z8, 8)
    val2, h1r = _layer1(acc1, val1, w1l8, w1r8, b1r, W2l, W2r, b2r)
    acc2 = _segsum_sc(val2, src_i, dst_i, z32, H4)
    r = _final(acc2, acc1, h1r, batchcol, cw1d, cb1d, cw2d, cb2d, seg16)
    return r.reshape(G, 16)[:, :OUT]
